# baseline (speedup 1.0000x reference)
import functools

import jax
import jax.numpy as jnp
from jax import lax
from jax.experimental import pallas as pl
from jax.experimental.pallas import tpu as pltpu
from jax.experimental.pallas import tpu_sc as plsc

_GAMMA = 1.0
_NEG_INF = float("-inf")

_BC = 2048
_NW = 32
_LANES = 16
_CCH = 40
_NCHUNK = 20
_WSUB = _CCH * _NCHUNK
_WSC = _WSUB * _NW


def _tc_body(y_ref, xt_ref, m_ref, c_ref, *, bc, ctc, nsteps, nb):
    c = pl.program_id(0)

    @pl.when(c == 0)
    def _init():
        m_ref[...] = jnp.full((1, nb), _NEG_INF, dtype=jnp.float32)
        c_ref[...] = jnp.zeros((1, nb), dtype=jnp.float32)

    xb = xt_ref[...]
    li = jax.lax.broadcasted_iota(jnp.int32, (bc, nb), 0)
    y_loc = y_ref[...] - c * bc
    eq = li == y_loc

    @pl.when(c < nsteps - 1)
    def _main():
        masked = jnp.where(eq, _NEG_INF, xb)
        m_ref[...] = jnp.maximum(
            m_ref[...], jnp.max(masked, axis=0, keepdims=True)
        )
        c_ref[...] = c_ref[...] + jnp.sum(
            jnp.where(eq, xb, 0.0), axis=0, keepdims=True
        )

    @pl.when(c == nsteps - 1)
    def _tail():
        oob = li >= (ctc - c * bc)
        masked = jnp.where(eq | oob, _NEG_INF, xb)
        m_ref[...] = jnp.maximum(
            m_ref[...], jnp.max(masked, axis=0, keepdims=True)
        )
        c_ref[...] = c_ref[...] + jnp.sum(
            jnp.where(eq & jnp.logical_not(oob), xb, 0.0),
            axis=0,
            keepdims=True,
        )


def _sc_body(xt_hbm, y_hbm, m_out, c_out, y_v, buf, m_v, c_v, *, ctc, nb):
    cid = lax.axis_index("c")
    sid = lax.axis_index("s")
    wid = sid * 2 + cid
    cls0 = ctc + wid * _WSUB
    nvr = nb // _LANES

    pltpu.sync_copy(y_hbm, y_v)

    neg = jnp.full((_LANES,), _NEG_INF, dtype=jnp.float32)

    def init_body(i, carry):
        m_v[pl.ds(i * _LANES, _LANES)] = neg
        c_v[pl.ds(i * _LANES, _LANES)] = neg
        return carry

    lax.fori_loop(0, nvr, init_body, 0)

    def chunk_body(ch, carry):
        cc0 = pl.multiple_of(cls0 + ch * _CCH, 8)
        pltpu.sync_copy(xt_hbm.at[pl.ds(cc0, _CCH), :], buf)

        for rt in range(8):
            ro = rt * 128
            y8 = [y_v[pl.ds(ro + t * _LANES, _LANES)] for t in range(8)]
            acc0 = tuple(
                m_v[pl.ds(ro + t * _LANES, _LANES)] for t in range(8)
            )
            cacc0 = tuple(
                c_v[pl.ds(ro + t * _LANES, _LANES)] for t in range(8)
            )

            def cls_body(cc, mc, ro=ro, y8=y8):
                accs, caccs = mc
                cb = jnp.zeros((_LANES,), dtype=jnp.int32) + (cc0 + cc)
                na, ncc = [], []
                for t in range(8):
                    v = buf[cc, pl.ds(ro + t * _LANES, _LANES)]
                    eq = y8[t] == cb
                    na.append(jnp.maximum(accs[t], jnp.where(eq, _NEG_INF, v)))
                    ncc.append(jnp.maximum(caccs[t], jnp.where(eq, v, _NEG_INF)))
                return (tuple(na), tuple(ncc))

            accs, caccs = lax.fori_loop(0, _CCH, cls_body, (acc0, cacc0))
            for t in range(8):
                m_v[pl.ds(ro + t * _LANES, _LANES)] = accs[t]
                c_v[pl.ds(ro + t * _LANES, _LANES)] = caccs[t]
        return carry

    lax.fori_loop(0, _NCHUNK, chunk_body, 0)

    pltpu.sync_copy(m_v, m_out.at[pl.ds(wid * nb, nb)])
    pltpu.sync_copy(c_v, c_out.at[pl.ds(wid * nb, nb)])


def _combine_body(mt_ref, ct_ref, ms_ref, cs_ref, o_ref, *, nb):
    ms = jnp.max(ms_ref[...], axis=0, keepdims=True)
    cv = jnp.max(cs_ref[...], axis=0, keepdims=True)
    cs = jnp.where(cv == _NEG_INF, 0.0, cv)
    m = jnp.maximum(mt_ref[...], ms)
    corr = ct_ref[...] + cs
    loss = _GAMMA + m - corr
    o_ref[0, 0] = jnp.sum(loss) * (1.0 / nb)


def kernel(x, y):
    nb, ncls = x.shape
    xt = x.T
    ctc = ncls - _WSC
    y32 = y.astype(jnp.int32)

    mesh = plsc.VectorSubcoreMesh(core_axis_name="c", subcore_axis_name="s")
    sc_fn = pl.kernel(
        functools.partial(_sc_body, ctc=ctc, nb=nb),
        mesh=mesh,
        out_type=[
            jax.ShapeDtypeStruct((_NW * nb,), jnp.float32),
            jax.ShapeDtypeStruct((_NW * nb,), jnp.float32),
        ],
        scratch_types=[
            pltpu.VMEM((nb,), jnp.int32),
            pltpu.VMEM((_CCH, nb), jnp.float32),
            pltpu.VMEM((nb,), jnp.float32),
            pltpu.VMEM((nb,), jnp.float32),
        ],
    )
    m_sc, c_sc = sc_fn(xt, y32)

    nsteps = pl.cdiv(ctc, _BC)
    tc_body = functools.partial(
        _tc_body, bc=_BC, ctc=ctc, nsteps=nsteps, nb=nb
    )
    m_tc, c_tc = pl.pallas_call(
        tc_body,
        grid=(nsteps,),
        in_specs=[
            pl.BlockSpec((1, nb), lambda c: (0, 0)),
            pl.BlockSpec((_BC, nb), lambda c: (c, 0)),
        ],
        out_specs=[
            pl.BlockSpec((1, nb), lambda c: (0, 0)),
            pl.BlockSpec((1, nb), lambda c: (0, 0)),
        ],
        out_shape=[
            jax.ShapeDtypeStruct((1, nb), jnp.float32),
            jax.ShapeDtypeStruct((1, nb), jnp.float32),
        ],
        compiler_params=pltpu.CompilerParams(
            dimension_semantics=("arbitrary",),
        ),
    )(y32.reshape(1, nb), xt)

    combine = functools.partial(_combine_body, nb=nb)
    out = pl.pallas_call(
        combine,
        out_specs=pl.BlockSpec(memory_space=pltpu.SMEM),
        out_shape=jax.ShapeDtypeStruct((1, 1), jnp.float32),
    )(
        m_tc,
        c_tc,
        m_sc.reshape(_NW, nb),
        c_sc.reshape(_NW, nb),
    )
    return out[0, 0]
